# Initial kernel scaffold; baseline (speedup 1.0000x reference)
import jax
import jax.numpy as jnp
from jax.experimental import pallas as pl


def kernel(ind_2, p1, p3, diff, basis, W_pp1_0, b_pp1_0, W_pp1_1, b_pp1_1, W_pi_0, b_pi_0, W_pi_1, b_pi_1, W_ii_0, W_ii_1, W_pp3_0, W_pp3_1, W_pix_i, W_pix_j, W_dot_i, W_dot_j):
    raise NotImplementedError("write your pallas kernel here")


# speedup vs baseline: 32.9791x; 32.9791x over previous
import functools

import jax
import jax.numpy as jnp
import numpy as np
from jax import lax
from jax.experimental import pallas as pl
from jax.experimental.pallas import tpu as pltpu
from jax.experimental.pallas import tpu_sc as plsc

N_ATOMS = 50000
N_PAIRS = 1600000
D = 16

BA = 2000
BP = 2000

NC = 2
NS = 16
NW = NC * NS

GCHUNK = 1000
SCHUNK = 400
ZROWS = 400

_INTERPRET = False
_USE_SC = True


def _tc1_body(p1_ref, p348_ref,
              W0_ref, b0_ref, W1_ref, b1_ref, Wpi_a_ref, Wpi_b_ref,
              Wpp3_ref, Wpix_i_ref, Wpix_j_ref,
              ta_ref, tb_ref):
    f32 = jnp.float32

    def mm(a, b):
        return jnp.dot(a, b, preferred_element_type=f32)

    p1 = p1_ref[...]
    h = jnp.tanh(mm(p1, W0_ref[...]) + b0_ref[...])
    p1a = jnp.tanh(mm(h, W1_ref[...]) + b1_ref[...])
    A = mm(p1a, Wpi_a_ref[...])
    B = mm(p1a, Wpi_b_ref[...])
    Wci = mm(Wpp3_ref[...], Wpix_i_ref[...])
    Wcj = mm(Wpp3_ref[...], Wpix_j_ref[...])
    p348 = p348_ref[...]
    pi_parts = [A]
    pj_parts = [B]
    for x in range(3):
        blk = p348[:, x * D:(x + 1) * D]
        pi_parts.append(mm(blk, Wci))
        pj_parts.append(mm(blk, Wcj))
    ta_ref[...] = jnp.concatenate(pi_parts, axis=1)
    tb_ref[...] = jnp.concatenate(pj_parts, axis=1)


def _tc1(p1, p348, W0, b0, W1, b1, Wpi_a, Wpi_b, Wpp3c, Wpix_i, Wpix_j):
    n = p1.shape[0]
    grid = n // BA
    row = lambda i: (i, 0)
    w_spec = pl.BlockSpec((D, D), lambda i: (0, 0))
    b_spec = pl.BlockSpec((1, D), lambda i: (0, 0))
    return pl.pallas_call(
        _tc1_body,
        grid=(grid,),
        in_specs=[
            pl.BlockSpec((BA, D), row),
            pl.BlockSpec((BA, 3 * D), row),
            w_spec, b_spec, w_spec, b_spec, w_spec, w_spec,
            w_spec, w_spec, w_spec,
        ],
        out_specs=[
            pl.BlockSpec((BA, 4 * D), row),
            pl.BlockSpec((BA, 4 * D), row),
        ],
        out_shape=[
            jax.ShapeDtypeStruct((n, 4 * D), jnp.float32),
            jax.ShapeDtypeStruct((n, 4 * D), jnp.float32),
        ],
        interpret=_INTERPRET,
    )(p1, p348, W0, b0, W1, b1, Wpi_a, Wpi_b, Wpp3c, Wpix_i, Wpix_j)


def _sc_gather(ta, tb, idx_i, idx_j):
    n_pairs = idx_i.shape[0]
    per_w = n_pairs // NW
    iters = per_w // GCHUNK
    mesh = plsc.VectorSubcoreMesh(core_axis_name="c", subcore_axis_name="s")

    @functools.partial(
        pl.kernel,
        out_type=[jax.ShapeDtypeStruct((n_pairs, 4 * D), jnp.float32),
                  jax.ShapeDtypeStruct((n_pairs, 4 * D), jnp.float32)],
        mesh=mesh,
        scratch_types=[
            pltpu.VMEM((GCHUNK,), jnp.int32),
            pltpu.VMEM((GCHUNK,), jnp.int32),
            pltpu.VMEM((GCHUNK, 4 * D), jnp.float32),
            pltpu.VMEM((GCHUNK, 4 * D), jnp.float32),
            pltpu.SemaphoreType.DMA,
            pltpu.SemaphoreType.DMA,
        ],
        compiler_params=pltpu.CompilerParams(use_tc_tiling_on_sc=False),
    )
    def k(ta_hbm, tb_hbm, ii_hbm, ij_hbm, gi_hbm, gj_hbm,
          ii_v, ij_v, ri_v, rj_v, sem1, sem2):
        wid = lax.axis_index("s") * NC + lax.axis_index("c")
        base0 = wid * per_w

        def body(it, carry):
            base = pl.multiple_of(base0 + it * GCHUNK, 8)
            pltpu.sync_copy(ii_hbm.at[pl.ds(base, GCHUNK)], ii_v)
            pltpu.sync_copy(ij_hbm.at[pl.ds(base, GCHUNK)], ij_v)
            c1 = pltpu.async_copy(ta_hbm.at[ii_v], ri_v, sem1)
            c2 = pltpu.async_copy(tb_hbm.at[ij_v], rj_v, sem2)
            c1.wait()
            c2.wait()
            pltpu.sync_copy(ri_v, gi_hbm.at[pl.ds(base, GCHUNK)])
            pltpu.sync_copy(rj_v, gj_hbm.at[pl.ds(base, GCHUNK)])
            return carry

        lax.fori_loop(0, iters, body, 0)

    return k(ta, tb, idx_i, idx_j)


def _tc2_body(gi_ref, gj_ref, diff_ref, basis_ref,
              Wpi1_ref, bpi0_ref, bpi1_ref, W2_ref, Wii1_ref,
              E1_ref, TB_ref, TD_ref, M1_ref, M3_ref,
              UO_ref, UI_ref, U1_ref,
              u0_ref, u1_ref):
    f32 = jnp.float32

    def mm(a, b):
        return jnp.dot(a, b, preferred_element_type=f32)

    g = gi_ref[...] + gj_ref[...]
    t1 = jnp.tanh(mm(g, E1_ref[...]) + bpi0_ref[...])
    inter = jnp.tanh(mm(t1, Wpi1_ref[...]) + bpi1_ref[...])
    bt = mm(basis_ref[...], TB_ref[...])
    z = inter * bt
    t2 = jnp.tanh(mm(z, W2_ref[...]))
    i1f = jnp.tanh(mm(t2, Wii1_ref[...]))
    dexp = mm(diff_ref[...], TD_ref[...])
    a3 = mm(i1f, M3_ref[...])
    a1 = mm(i1f, M1_ref[...])
    i3_64 = g * a3 + dexp * a1
    u0_ref[...] = mm(i1f, UO_ref[...]) + mm(i3_64, UI_ref[...])
    u1_ref[...] = mm(i3_64, U1_ref[...])


def _tc2(gi, gj, diff, basis, Wpi1, bpi0, bpi1, W2, Wii1, consts):
    n = gi.shape[0]
    grid = n // BP
    row = lambda i: (i, 0)

    def full(a):
        return pl.BlockSpec(a.shape, lambda i: (0,) * a.ndim)

    return pl.pallas_call(
        _tc2_body,
        grid=(grid,),
        in_specs=[
            pl.BlockSpec((BP, 4 * D), row),
            pl.BlockSpec((BP, 4 * D), row),
            pl.BlockSpec((BP, 3), row),
            pl.BlockSpec((BP, 4), row),
            full(Wpi1), full(bpi0), full(bpi1), full(W2), full(Wii1),
        ] + [full(c) for c in consts],
        out_specs=[
            pl.BlockSpec((BP, 2 * D), row),
            pl.BlockSpec((BP, 2 * D), row),
        ],
        out_shape=[
            jax.ShapeDtypeStruct((n, 2 * D), jnp.float32),
            jax.ShapeDtypeStruct((n, 2 * D), jnp.float32),
        ],
        interpret=_INTERPRET,
    )(gi, gj, diff, basis, Wpi1, bpi0, bpi1, W2, Wii1, *consts)


def _tc2_consts():
    E1 = np.zeros((4 * D, D), np.float32)
    for c in range(D):
        E1[c, c] = 1.0
    TB = np.zeros((4, 4 * D), np.float32)
    for l in range(4 * D):
        TB[l % 4, l] = 1.0
    TD = np.zeros((3, 4 * D), np.float32)
    for l in range(3 * D):
        TD[l // D, D + l] = 1.0
    M1 = np.zeros((3 * D, 4 * D), np.float32)
    M3 = np.zeros((3 * D, 4 * D), np.float32)
    for l in range(3 * D):
        M1[l % D, D + l] = 1.0
        M3[2 * D + l % D, D + l] = 1.0
    UO = np.zeros((3 * D, 2 * D), np.float32)
    for c in range(D):
        UO[D + c, c] = 1.0
    UI = np.zeros((4 * D, 2 * D), np.float32)
    for c in range(D):
        UI[D + c, D + c] = 1.0
    U1 = np.zeros((4 * D, 2 * D), np.float32)
    for c in range(2 * D):
        U1[2 * D + c, c] = 1.0
    return [jnp.asarray(x) for x in (E1, TB, TD, M1, M3, UO, UI, U1)]


def _sc_scatter(u0, u1, idx_i):
    n_pairs = idx_i.shape[0]
    per_tile = n_pairs // NS
    iters = per_tile // SCHUNK
    n_chunks = N_ATOMS // ZROWS
    chunks_per_tile = (n_chunks + NS - 1) // NS
    mesh = plsc.VectorSubcoreMesh(core_axis_name="c", subcore_axis_name="s")

    @functools.partial(
        pl.kernel,
        out_type=[jax.ShapeDtypeStruct((N_ATOMS, 2 * D), jnp.float32),
                  jax.ShapeDtypeStruct((N_ATOMS, 2 * D), jnp.float32)],
        mesh=mesh,
        scratch_types=[
            pltpu.VMEM((SCHUNK,), jnp.int32),
            pltpu.VMEM((SCHUNK,), jnp.int32),
            pltpu.VMEM((SCHUNK, 2 * D), jnp.float32),
            pltpu.VMEM((SCHUNK, 2 * D), jnp.float32),
            pltpu.VMEM_SHARED((N_ATOMS, 2 * D), jnp.float32),
            pltpu.SemaphoreType.DMA,
            pltpu.SemaphoreType.DMA,
        ],
        compiler_params=pltpu.CompilerParams(use_tc_tiling_on_sc=False),
    )
    def k(u0_hbm, u1_hbm, idx_hbm, out0_hbm, out1_hbm,
          idx0, idx1, val0, val1, acc_sh, p0, p1):
        c = lax.axis_index("c")
        s = lax.axis_index("s")
        idxb = (idx0, idx1)
        valb = (val0, val1)
        psem = (p0, p1)

        zero = jnp.zeros((D,), jnp.float32)

        def zbody(r, carry):
            val0[r, 0:D] = zero
            val0[r, D:2 * D] = zero
            return carry

        lax.fori_loop(0, SCHUNK, zbody, 0)

        def zchunk(kk, carry):
            ch = s + kk * NS

            @pl.when(ch < n_chunks)
            def _():
                r0 = pl.multiple_of(ch * ZROWS, 8)
                pltpu.sync_copy(val0, acc_sh.at[pl.ds(r0, ZROWS)])

            return carry

        lax.fori_loop(0, chunks_per_tile, zchunk, 0)
        plsc.subcore_barrier()

        def scatter_all(u_hbm):
            def run(it, carry):
                base = pl.multiple_of(s * per_tile + it * SCHUNK, 8)
                pltpu.sync_copy(idx_hbm.at[pl.ds(base, SCHUNK)], idx0)
                pltpu.sync_copy(u_hbm.at[pl.ds(base, SCHUNK)], val1)
                pltpu.sync_copy(val1, acc_sh.at[idx0], add=True)
                return carry

            lax.fori_loop(0, iters, run, 0)

        @pl.when(c == 0)
        def _():
            scatter_all(u0_hbm)

        @pl.when(c == 1)
        def _():
            scatter_all(u1_hbm)

        plsc.subcore_barrier()

        def copy_out(out_hbm):
            def cchunk(kk, carry):
                ch = s + kk * NS

                @pl.when(ch < n_chunks)
                def _():
                    r0 = pl.multiple_of(ch * ZROWS, 8)
                    pltpu.sync_copy(acc_sh.at[pl.ds(r0, ZROWS)], val0)
                    pltpu.sync_copy(val0, out_hbm.at[pl.ds(r0, ZROWS)])

                return carry

            lax.fori_loop(0, chunks_per_tile, cchunk, 0)

        @pl.when(c == 0)
        def _():
            copy_out(out0_hbm)

        @pl.when(c == 1)
        def _():
            copy_out(out1_hbm)

    return k(u0, u1, idx_i)


def _tc3_body(acc0_ref, acc1_ref, Wd_i_ref, Wd_j_ref, p1t_ref, p3t_ref):
    f32 = jnp.float32

    def mm(a, b):
        return jnp.dot(a, b, preferred_element_type=f32)

    acc0 = acc0_ref[...]
    acc1 = acc1_ref[...]
    p1b = acc0[:, :D]
    Wd_i = Wd_i_ref[...]
    Wd_j = Wd_j_ref[...]
    blks = [acc0[:, D:2 * D], acc1[:, :D], acc1[:, D:2 * D]]
    dot = jnp.zeros_like(p1b)
    for blk in blks:
        dot = dot + mm(blk, Wd_i) * mm(blk, Wd_j)
    p1t = dot + p1b
    p1t_ref[...] = p1t
    p3t_ref[...] = jnp.concatenate([b * p1t for b in blks], axis=1)


def _tc3(acc0, acc1, Wd_i, Wd_j):
    n = acc0.shape[0]
    grid = n // BA
    row = lambda i: (i, 0)
    w_spec = pl.BlockSpec((D, D), lambda i: (0, 0))
    return pl.pallas_call(
        _tc3_body,
        grid=(grid,),
        in_specs=[pl.BlockSpec((BA, 2 * D), row),
                  pl.BlockSpec((BA, 2 * D), row), w_spec, w_spec],
        out_specs=[
            pl.BlockSpec((BA, D), row),
            pl.BlockSpec((BA, 3 * D), row),
        ],
        out_shape=[
            jax.ShapeDtypeStruct((n, D), jnp.float32),
            jax.ShapeDtypeStruct((n, 3 * D), jnp.float32),
        ],
        interpret=_INTERPRET,
    )(acc0, acc1, Wd_i, Wd_j)


def kernel(ind_2, p1, p3, diff, basis, W_pp1_0, b_pp1_0, W_pp1_1, b_pp1_1,
           W_pi_0, b_pi_0, W_pi_1, b_pi_1, W_ii_0, W_ii_1, W_pp3_0, W_pp3_1,
           W_pix_i, W_pix_j, W_dot_i, W_dot_j):
    n_atoms = p1.shape[0]
    idx_i = ind_2[:, 0].astype(jnp.int32)
    idx_j = ind_2[:, 1].astype(jnp.int32)
    p348 = p3.reshape(n_atoms, 3 * D)
    hi = jax.lax.Precision.HIGHEST
    Wpp3c = jnp.dot(W_pp3_0, W_pp3_1, precision=hi)

    ta, tb = _tc1(p1, p348,
                  W_pp1_0, b_pp1_0.reshape(1, D), W_pp1_1, b_pp1_1.reshape(1, D),
                  W_pi_0[:D], W_pi_0[D:], Wpp3c, W_pix_i, W_pix_j)

    if _USE_SC:
        gi, gj = _sc_gather(ta, tb, idx_i, idx_j)
    else:
        gi = jnp.take(ta, idx_i, axis=0)
        gj = jnp.take(tb, idx_j, axis=0)

    Sm = np.zeros((4 * D, D), np.float32)
    for l in range(4 * D):
        Sm[l, l // 4] = 1.0
    W2 = jnp.dot(jnp.asarray(Sm), W_ii_0, precision=hi)
    consts = _tc2_consts()
    u0, u1 = _tc2(gi, gj, diff, basis,
                  W_pi_1, b_pi_0.reshape(1, D), b_pi_1.reshape(1, 4 * D),
                  W2, W_ii_1, consts)

    if _USE_SC:
        acc0, acc1 = _sc_scatter(u0, u1, idx_i)
    else:
        acc0 = jax.ops.segment_sum(u0, idx_i, num_segments=n_atoms)
        acc1 = jax.ops.segment_sum(u1, idx_i, num_segments=n_atoms)

    p1t1, p3t48 = _tc3(acc0, acc1, W_dot_i, W_dot_j)
    return (p1t1, p3t48.reshape(n_atoms, 3, D))


# speedup vs baseline: 33.0120x; 1.0010x over previous
import functools

import jax
import jax.numpy as jnp
import numpy as np
from jax import lax
from jax.experimental import pallas as pl
from jax.experimental.pallas import tpu as pltpu
from jax.experimental.pallas import tpu_sc as plsc

N_ATOMS = 50000
N_PAIRS = 1600000
D = 16

BA = 2000
BP = 2000

NC = 2
NS = 16
NW = NC * NS

GCHUNK = 1000
SCHUNK = 400
ZROWS = 400


def _tc1_body(p1_ref, p348_ref,
              W0_ref, b0_ref, W1_ref, b1_ref, Wpi_a_ref, Wpi_b_ref,
              Wpp3_ref, Wpix_i_ref, Wpix_j_ref,
              ta_ref, tb_ref):
    f32 = jnp.float32

    def mm(a, b):
        return jnp.dot(a, b, preferred_element_type=f32)

    p1 = p1_ref[...]
    h = jnp.tanh(mm(p1, W0_ref[...]) + b0_ref[...])
    p1a = jnp.tanh(mm(h, W1_ref[...]) + b1_ref[...])
    A = mm(p1a, Wpi_a_ref[...])
    B = mm(p1a, Wpi_b_ref[...])
    Wci = mm(Wpp3_ref[...], Wpix_i_ref[...])
    Wcj = mm(Wpp3_ref[...], Wpix_j_ref[...])
    p348 = p348_ref[...]
    pi_parts = [A]
    pj_parts = [B]
    for x in range(3):
        blk = p348[:, x * D:(x + 1) * D]
        pi_parts.append(mm(blk, Wci))
        pj_parts.append(mm(blk, Wcj))
    ta_ref[...] = jnp.concatenate(pi_parts, axis=1)
    tb_ref[...] = jnp.concatenate(pj_parts, axis=1)


def _tc1(p1, p348, W0, b0, W1, b1, Wpi_a, Wpi_b, Wpp3c, Wpix_i, Wpix_j):
    n = p1.shape[0]
    grid = n // BA
    row = lambda i: (i, 0)
    w_spec = pl.BlockSpec((D, D), lambda i: (0, 0))
    b_spec = pl.BlockSpec((1, D), lambda i: (0, 0))
    return pl.pallas_call(
        _tc1_body,
        grid=(grid,),
        in_specs=[
            pl.BlockSpec((BA, D), row),
            pl.BlockSpec((BA, 3 * D), row),
            w_spec, b_spec, w_spec, b_spec, w_spec, w_spec,
            w_spec, w_spec, w_spec,
        ],
        out_specs=[
            pl.BlockSpec((BA, 4 * D), row),
            pl.BlockSpec((BA, 4 * D), row),
        ],
        out_shape=[
            jax.ShapeDtypeStruct((n, 4 * D), jnp.float32),
            jax.ShapeDtypeStruct((n, 4 * D), jnp.float32),
        ],
    )(p1, p348, W0, b0, W1, b1, Wpi_a, Wpi_b, Wpp3c, Wpix_i, Wpix_j)


def _sc_gather(ta, tb, idx_i, idx_j):
    n_pairs = idx_i.shape[0]
    per_w = n_pairs // NW
    iters = per_w // GCHUNK
    mesh = plsc.VectorSubcoreMesh(core_axis_name="c", subcore_axis_name="s")

    @functools.partial(
        pl.kernel,
        out_type=[jax.ShapeDtypeStruct((n_pairs, 4 * D), jnp.float32),
                  jax.ShapeDtypeStruct((n_pairs, 4 * D), jnp.float32)],
        mesh=mesh,
        scratch_types=[
            pltpu.VMEM((GCHUNK,), jnp.int32),
            pltpu.VMEM((GCHUNK,), jnp.int32),
            pltpu.VMEM((GCHUNK, 4 * D), jnp.float32),
            pltpu.VMEM((GCHUNK, 4 * D), jnp.float32),
            pltpu.SemaphoreType.DMA,
            pltpu.SemaphoreType.DMA,
        ],
        compiler_params=pltpu.CompilerParams(use_tc_tiling_on_sc=False),
    )
    def k(ta_hbm, tb_hbm, ii_hbm, ij_hbm, gi_hbm, gj_hbm,
          ii_v, ij_v, ri_v, rj_v, sem1, sem2):
        wid = lax.axis_index("s") * NC + lax.axis_index("c")
        base0 = wid * per_w

        def body(it, carry):
            base = pl.multiple_of(base0 + it * GCHUNK, 8)
            pltpu.sync_copy(ii_hbm.at[pl.ds(base, GCHUNK)], ii_v)
            pltpu.sync_copy(ij_hbm.at[pl.ds(base, GCHUNK)], ij_v)
            c1 = pltpu.async_copy(ta_hbm.at[ii_v], ri_v, sem1)
            c2 = pltpu.async_copy(tb_hbm.at[ij_v], rj_v, sem2)
            c1.wait()
            c2.wait()
            pltpu.sync_copy(ri_v, gi_hbm.at[pl.ds(base, GCHUNK)])
            pltpu.sync_copy(rj_v, gj_hbm.at[pl.ds(base, GCHUNK)])
            return carry

        lax.fori_loop(0, iters, body, 0)

    return k(ta, tb, idx_i, idx_j)


def _tc2_body(gi_ref, gj_ref, diff_ref, basis_ref,
              Wpi1_ref, bpi0_ref, bpi1_ref, W2_ref, Wii1_ref,
              E1_ref, TB_ref, TD_ref, M1_ref, M3_ref,
              UO_ref, UI_ref, U1_ref,
              u0_ref, u1_ref):
    f32 = jnp.float32

    def mm(a, b):
        return jnp.dot(a, b, preferred_element_type=f32)

    g = gi_ref[...] + gj_ref[...]
    t1 = jnp.tanh(mm(g, E1_ref[...]) + bpi0_ref[...])
    inter = jnp.tanh(mm(t1, Wpi1_ref[...]) + bpi1_ref[...])
    bt = mm(basis_ref[...], TB_ref[...])
    z = inter * bt
    t2 = jnp.tanh(mm(z, W2_ref[...]))
    i1f = jnp.tanh(mm(t2, Wii1_ref[...]))
    dexp = mm(diff_ref[...], TD_ref[...])
    a3 = mm(i1f, M3_ref[...])
    a1 = mm(i1f, M1_ref[...])
    i3_64 = g * a3 + dexp * a1
    u0_ref[...] = mm(i1f, UO_ref[...]) + mm(i3_64, UI_ref[...])
    u1_ref[...] = mm(i3_64, U1_ref[...])


def _tc2(gi, gj, diff, basis, Wpi1, bpi0, bpi1, W2, Wii1, consts):
    n = gi.shape[0]
    grid = n // BP
    row = lambda i: (i, 0)

    def full(a):
        return pl.BlockSpec(a.shape, lambda i: (0,) * a.ndim)

    return pl.pallas_call(
        _tc2_body,
        grid=(grid,),
        in_specs=[
            pl.BlockSpec((BP, 4 * D), row),
            pl.BlockSpec((BP, 4 * D), row),
            pl.BlockSpec((BP, 3), row),
            pl.BlockSpec((BP, 4), row),
            full(Wpi1), full(bpi0), full(bpi1), full(W2), full(Wii1),
        ] + [full(c) for c in consts],
        out_specs=[
            pl.BlockSpec((BP, 2 * D), row),
            pl.BlockSpec((BP, 2 * D), row),
        ],
        out_shape=[
            jax.ShapeDtypeStruct((n, 2 * D), jnp.float32),
            jax.ShapeDtypeStruct((n, 2 * D), jnp.float32),
        ],
    )(gi, gj, diff, basis, Wpi1, bpi0, bpi1, W2, Wii1, *consts)


def _tc2_consts():
    E1 = np.zeros((4 * D, D), np.float32)
    for c in range(D):
        E1[c, c] = 1.0
    TB = np.zeros((4, 4 * D), np.float32)
    for l in range(4 * D):
        TB[l % 4, l] = 1.0
    TD = np.zeros((3, 4 * D), np.float32)
    for l in range(3 * D):
        TD[l // D, D + l] = 1.0
    M1 = np.zeros((3 * D, 4 * D), np.float32)
    M3 = np.zeros((3 * D, 4 * D), np.float32)
    for l in range(3 * D):
        M1[l % D, D + l] = 1.0
        M3[2 * D + l % D, D + l] = 1.0
    UO = np.zeros((3 * D, 2 * D), np.float32)
    for c in range(D):
        UO[D + c, c] = 1.0
    UI = np.zeros((4 * D, 2 * D), np.float32)
    for c in range(D):
        UI[D + c, D + c] = 1.0
    U1 = np.zeros((4 * D, 2 * D), np.float32)
    for c in range(2 * D):
        U1[2 * D + c, c] = 1.0
    return [jnp.asarray(x) for x in (E1, TB, TD, M1, M3, UO, UI, U1)]


def _sc_scatter(u0, u1, idx_i):
    n_pairs = idx_i.shape[0]
    per_tile = n_pairs // NS
    iters = per_tile // SCHUNK
    n_chunks = N_ATOMS // ZROWS
    chunks_per_tile = (n_chunks + NS - 1) // NS
    mesh = plsc.VectorSubcoreMesh(core_axis_name="c", subcore_axis_name="s")

    @functools.partial(
        pl.kernel,
        out_type=[jax.ShapeDtypeStruct((N_ATOMS, 2 * D), jnp.float32),
                  jax.ShapeDtypeStruct((N_ATOMS, 2 * D), jnp.float32)],
        mesh=mesh,
        scratch_types=[
            pltpu.VMEM((SCHUNK,), jnp.int32),
            pltpu.VMEM((SCHUNK,), jnp.int32),
            pltpu.VMEM((SCHUNK, 2 * D), jnp.float32),
            pltpu.VMEM((SCHUNK, 2 * D), jnp.float32),
            pltpu.VMEM_SHARED((N_ATOMS, 2 * D), jnp.float32),
            pltpu.SemaphoreType.DMA,
            pltpu.SemaphoreType.DMA,
        ],
        compiler_params=pltpu.CompilerParams(use_tc_tiling_on_sc=False),
    )
    def k(u0_hbm, u1_hbm, idx_hbm, out0_hbm, out1_hbm,
          idx0, idx1, val0, val1, acc_sh, p0, p1):
        c = lax.axis_index("c")
        s = lax.axis_index("s")
        idxb = (idx0, idx1)
        valb = (val0, val1)
        psem = (p0, p1)

        zero = jnp.zeros((D,), jnp.float32)

        def zbody(r, carry):
            val0[r, 0:D] = zero
            val0[r, D:2 * D] = zero
            return carry

        lax.fori_loop(0, SCHUNK, zbody, 0)

        def zchunk(kk, carry):
            ch = s + kk * NS

            @pl.when(ch < n_chunks)
            def _():
                r0 = pl.multiple_of(ch * ZROWS, 8)
                pltpu.sync_copy(val0, acc_sh.at[pl.ds(r0, ZROWS)])

            return carry

        lax.fori_loop(0, chunks_per_tile, zchunk, 0)
        plsc.subcore_barrier()

        def scatter_all(u_hbm):
            def run(it, carry):
                base = pl.multiple_of(s * per_tile + it * SCHUNK, 8)
                pltpu.sync_copy(idx_hbm.at[pl.ds(base, SCHUNK)], idx0)
                pltpu.sync_copy(u_hbm.at[pl.ds(base, SCHUNK)], val1)
                pltpu.sync_copy(val1, acc_sh.at[idx0], add=True)
                return carry

            lax.fori_loop(0, iters, run, 0)

        @pl.when(c == 0)
        def _():
            scatter_all(u0_hbm)

        @pl.when(c == 1)
        def _():
            scatter_all(u1_hbm)

        plsc.subcore_barrier()

        def copy_out(out_hbm):
            def cchunk(kk, carry):
                ch = s + kk * NS

                @pl.when(ch < n_chunks)
                def _():
                    r0 = pl.multiple_of(ch * ZROWS, 8)
                    pltpu.sync_copy(acc_sh.at[pl.ds(r0, ZROWS)], val0)
                    pltpu.sync_copy(val0, out_hbm.at[pl.ds(r0, ZROWS)])

                return carry

            lax.fori_loop(0, chunks_per_tile, cchunk, 0)

        @pl.when(c == 0)
        def _():
            copy_out(out0_hbm)

        @pl.when(c == 1)
        def _():
            copy_out(out1_hbm)

    return k(u0, u1, idx_i)


def _tc3_body(acc0_ref, acc1_ref, Wd_i_ref, Wd_j_ref, p1t_ref, p3t_ref):
    f32 = jnp.float32

    def mm(a, b):
        return jnp.dot(a, b, preferred_element_type=f32)

    acc0 = acc0_ref[...]
    acc1 = acc1_ref[...]
    p1b = acc0[:, :D]
    Wd_i = Wd_i_ref[...]
    Wd_j = Wd_j_ref[...]
    blks = [acc0[:, D:2 * D], acc1[:, :D], acc1[:, D:2 * D]]
    dot = jnp.zeros_like(p1b)
    for blk in blks:
        dot = dot + mm(blk, Wd_i) * mm(blk, Wd_j)
    p1t = dot + p1b
    p1t_ref[...] = p1t
    p3t_ref[...] = jnp.concatenate([b * p1t for b in blks], axis=1)


def _tc3(acc0, acc1, Wd_i, Wd_j):
    n = acc0.shape[0]
    grid = n // BA
    row = lambda i: (i, 0)
    w_spec = pl.BlockSpec((D, D), lambda i: (0, 0))
    return pl.pallas_call(
        _tc3_body,
        grid=(grid,),
        in_specs=[pl.BlockSpec((BA, 2 * D), row),
                  pl.BlockSpec((BA, 2 * D), row), w_spec, w_spec],
        out_specs=[
            pl.BlockSpec((BA, D), row),
            pl.BlockSpec((BA, 3 * D), row),
        ],
        out_shape=[
            jax.ShapeDtypeStruct((n, D), jnp.float32),
            jax.ShapeDtypeStruct((n, 3 * D), jnp.float32),
        ],
    )(acc0, acc1, Wd_i, Wd_j)


def kernel(ind_2, p1, p3, diff, basis, W_pp1_0, b_pp1_0, W_pp1_1, b_pp1_1,
           W_pi_0, b_pi_0, W_pi_1, b_pi_1, W_ii_0, W_ii_1, W_pp3_0, W_pp3_1,
           W_pix_i, W_pix_j, W_dot_i, W_dot_j):
    n_atoms = p1.shape[0]
    idx_i = ind_2[:, 0].astype(jnp.int32)
    idx_j = ind_2[:, 1].astype(jnp.int32)
    p348 = p3.reshape(n_atoms, 3 * D)
    hi = jax.lax.Precision.HIGHEST
    Wpp3c = jnp.dot(W_pp3_0, W_pp3_1, precision=hi)

    ta, tb = _tc1(p1, p348,
                  W_pp1_0, b_pp1_0.reshape(1, D), W_pp1_1, b_pp1_1.reshape(1, D),
                  W_pi_0[:D], W_pi_0[D:], Wpp3c, W_pix_i, W_pix_j)

    gi, gj = _sc_gather(ta, tb, idx_i, idx_j)

    Sm = np.zeros((4 * D, D), np.float32)
    for l in range(4 * D):
        Sm[l, l // 4] = 1.0
    W2 = jnp.dot(jnp.asarray(Sm), W_ii_0, precision=hi)
    consts = _tc2_consts()
    u0, u1 = _tc2(gi, gj, diff, basis,
                  W_pi_1, b_pi_0.reshape(1, D), b_pi_1.reshape(1, 4 * D),
                  W2, W_ii_1, consts)

    acc0, acc1 = _sc_scatter(u0, u1, idx_i)

    p1t1, p3t48 = _tc3(acc0, acc1, W_dot_i, W_dot_j)
    return (p1t1, p3t48.reshape(n_atoms, 3, D))
